# Initial kernel scaffold; baseline (speedup 1.0000x reference)
import jax
import jax.numpy as jnp
from jax.experimental import pallas as pl


def kernel(x, edge_index, W1, b1, W2, b2):
    raise NotImplementedError("write your pallas kernel here")


# speedup vs baseline: 9.3952x; 9.3952x over previous
import functools

import jax
import jax.numpy as jnp
from jax import lax
from jax.experimental import pallas as pl
from jax.experimental.pallas import tpu as pltpu
from jax.experimental.pallas import tpu_sc as plsc

N_NODES = 10000
D = 128

NC = 2
NS = 16
NW = NC * NS

CHUNK = 128
ROWS_PER_TILE = 640
N_PAD = ROWS_PER_TILE * NS

_MESH = plsc.VectorSubcoreMesh(core_axis_name="c", subcore_axis_name="s",
                               num_cores=NC, num_subcores=NS)


def _zero_fill(buf, nrows, ncols):
    zeros16 = jnp.zeros((16,), jnp.float32)

    def row(i, _):
        for j in range(ncols // 16):
            buf[i, pl.ds(j * 16, 16)] = zeros16
        return 0

    lax.fori_loop(0, nrows, row, 0)


def _copy_zero_to_slice(zbuf, dst, row0, nrows):
    zrows = zbuf.shape[0]
    off = 0
    while off < nrows:
        n = min(zrows, nrows - off)
        pltpu.sync_copy(zbuf.at[pl.ds(0, n)], dst.at[pl.ds(row0 + off, n)])
        off += n


def _sc_degree_body(dst_hbm, out_hbm, dsts_v, hist_v, red_v, stage_sh, sem,
                    *, chunks_per_tile):
    c = lax.axis_index("c")
    s = lax.axis_index("s")
    wid = c * NS + s

    cp = pltpu.async_copy(dst_hbm.at[wid], dsts_v, sem)

    def zrow(i, _):
        hist_v[pl.ds(i * 16, 16)] = jnp.zeros((16,), jnp.float32)
        return 0

    lax.fori_loop(0, N_PAD // 16, zrow, 0)
    cp.wait()

    ones16 = jnp.ones((16,), jnp.float32)

    def body(i, _):
        for j in range(CHUNK // 16):
            plsc.addupdate_scatter(hist_v, [dsts_v[i, pl.ds(j * 16, 16)]],
                                   ones16)
        return 0

    lax.fori_loop(0, chunks_per_tile, body, 0)

    pltpu.sync_copy(hist_v, stage_sh.at[pl.ds(s * N_PAD, N_PAD)])
    plsc.subcore_barrier()

    row0 = s * ROWS_PER_TILE
    for k in range(NS):
        pltpu.sync_copy(stage_sh.at[pl.ds(k * N_PAD + row0, ROWS_PER_TILE)],
                        red_v.at[pl.ds(k * ROWS_PER_TILE, ROWS_PER_TILE)])

    def red_row(j, _):
        tot = red_v[pl.ds(j * 16, 16)]
        for k in range(1, NS):
            tot = tot + red_v[pl.ds(k * ROWS_PER_TILE + j * 16, 16)]
        hist_v[pl.ds(j * 16, 16)] = tot
        return 0

    lax.fori_loop(0, ROWS_PER_TILE // 16, red_row, 0)
    pltpu.sync_copy(hist_v.at[pl.ds(0, ROWS_PER_TILE)],
                    out_hbm.at[pl.ds(c * N_PAD + row0, ROWS_PER_TILE)])


def _sc_segment_sum_body(g_hbm, src_hbm, dst_hbm, out_hbm,
                         srcs_v, dst0_v, dst1_v, rows0_v, rows1_v, acc_sh,
                         sem_i, sem0, sem1, semd0, semd1,
                         *, chunks_per_tile):
    c = lax.axis_index("c")
    s = lax.axis_index("s")
    wid = c * NS + s

    cp_s = pltpu.async_copy(src_hbm.at[wid], srcs_v, sem_i)

    _zero_fill(rows0_v, CHUNK, D)
    _copy_zero_to_slice(rows0_v, acc_sh, s * ROWS_PER_TILE, ROWS_PER_TILE)
    cp_s.wait()
    plsc.subcore_barrier()

    ebase = wid * chunks_per_tile * CHUNK

    pltpu.async_copy(dst_hbm.at[pl.ds(ebase, CHUNK)], dst0_v, semd0)
    pltpu.async_copy(dst_hbm.at[pl.ds(ebase + CHUNK, CHUNK)], dst1_v, semd1)
    pltpu.async_copy(g_hbm.at[srcs_v.at[0]], rows0_v, sem0)

    def pair(j, _):
        i0 = 2 * j
        pltpu.async_copy(g_hbm.at[srcs_v.at[i0 + 1]], rows1_v, sem1)
        pltpu.make_async_copy(dst_hbm.at[pl.ds(ebase, CHUNK)], dst0_v,
                              semd0).wait()
        pltpu.make_async_copy(g_hbm.at[srcs_v.at[i0]], rows0_v, sem0).wait()
        pltpu.sync_copy(rows0_v, acc_sh.at[dst0_v], add=True)

        @pl.when(i0 + 2 < chunks_per_tile)
        def _():
            pltpu.async_copy(g_hbm.at[srcs_v.at[i0 + 2]], rows0_v, sem0)
            pltpu.async_copy(dst_hbm.at[pl.ds(ebase + (i0 + 2) * CHUNK, CHUNK)],
                             dst0_v, semd0)

        pltpu.make_async_copy(dst_hbm.at[pl.ds(ebase, CHUNK)], dst1_v,
                              semd1).wait()
        pltpu.make_async_copy(g_hbm.at[srcs_v.at[i0 + 1]], rows1_v, sem1).wait()
        pltpu.sync_copy(rows1_v, acc_sh.at[dst1_v], add=True)

        @pl.when(i0 + 3 < chunks_per_tile)
        def _():
            pltpu.async_copy(dst_hbm.at[pl.ds(ebase + (i0 + 3) * CHUNK, CHUNK)],
                             dst1_v, semd1)

        return 0

    lax.fori_loop(0, chunks_per_tile // 2, pair, 0)
    plsc.subcore_barrier()

    row0 = s * ROWS_PER_TILE
    pltpu.sync_copy(acc_sh.at[pl.ds(row0, ROWS_PER_TILE)],
                    out_hbm.at[c].at[pl.ds(row0, ROWS_PER_TILE)])


def _sc_degree(dst_pad, chunks_per_tile):
    kfn = functools.partial(
        pl.kernel,
        out_type=jax.ShapeDtypeStruct((NC * N_PAD,), jnp.float32),
        mesh=_MESH,
        compiler_params=pltpu.CompilerParams(needs_layout_passes=False),
        scratch_types=[
            pltpu.VMEM((chunks_per_tile, CHUNK), jnp.int32),
            pltpu.VMEM((N_PAD,), jnp.float32),
            pltpu.VMEM((NS * ROWS_PER_TILE,), jnp.float32),
            pltpu.VMEM_SHARED((NS * N_PAD,), jnp.float32),
            pltpu.SemaphoreType.DMA,
        ],
    )(functools.partial(_sc_degree_body, chunks_per_tile=chunks_per_tile))
    return kfn(dst_pad)


def _sc_segment_sum(g, src3d, dst_flat, chunks_per_tile):
    kfn = functools.partial(
        pl.kernel,
        out_type=jax.ShapeDtypeStruct((NC, N_PAD, D), jnp.float32),
        mesh=_MESH,
        scratch_types=[
            pltpu.VMEM((chunks_per_tile, CHUNK), jnp.int32),
            pltpu.VMEM((CHUNK,), jnp.int32),
            pltpu.VMEM((CHUNK,), jnp.int32),
            pltpu.VMEM((CHUNK, D), jnp.float32),
            pltpu.VMEM((CHUNK, D), jnp.float32),
            pltpu.VMEM_SHARED((N_PAD, D), jnp.float32),
            pltpu.SemaphoreType.DMA,
            pltpu.SemaphoreType.DMA,
            pltpu.SemaphoreType.DMA,
            pltpu.SemaphoreType.DMA,
            pltpu.SemaphoreType.DMA,
        ],
    )(functools.partial(_sc_segment_sum_body, chunks_per_tile=chunks_per_tile))
    return kfn(g, src3d, dst_flat)


_BM = 1000


def _mm_body(x_ref, w_ref, o_ref):
    o_ref[...] = jnp.dot(x_ref[...], w_ref[...],
                         preferred_element_type=jnp.float32)


def _tc_matmul(x, w):
    n = x.shape[0]
    return pl.pallas_call(
        _mm_body,
        grid=(n // _BM,),
        in_specs=[
            pl.BlockSpec((_BM, D), lambda i: (i, 0)),
            pl.BlockSpec((D, D), lambda i: (0, 0)),
        ],
        out_specs=pl.BlockSpec((_BM, D), lambda i: (i, 0)),
        out_shape=jax.ShapeDtypeStruct((n, D), jnp.float32),
    )(x, w)


def _scale_body(h_ref, d0_ref, d1_ref, g_ref):
    dinv = lax.rsqrt(d0_ref[...] + d1_ref[...] + 1.0)
    g_ref[...] = h_ref[...] * dinv


def _tc_scale(h, deg0, deg1):
    n = h.shape[0]
    return pl.pallas_call(
        _scale_body,
        grid=(n // _BM,),
        in_specs=[
            pl.BlockSpec((_BM, D), lambda i: (i, 0)),
            pl.BlockSpec((_BM, 1), lambda i: (i, 0)),
            pl.BlockSpec((_BM, 1), lambda i: (i, 0)),
        ],
        out_specs=pl.BlockSpec((_BM, D), lambda i: (i, 0)),
        out_shape=jax.ShapeDtypeStruct((n, D), jnp.float32),
    )(h, deg0, deg1)


def _mid_body(p0_ref, p1_ref, g_ref, d0_ref, d1_ref, b_ref, w_ref, o_ref):
    dinv = lax.rsqrt(d0_ref[...] + d1_ref[...] + 1.0)
    t = (p0_ref[...] + p1_ref[...] + g_ref[...]) * dinv + b_ref[...]
    t = jnp.maximum(t, 0.0)
    o_ref[...] = jnp.dot(t, w_ref[...],
                         preferred_element_type=jnp.float32) * dinv


def _tc_mid(p0, p1, g, deg0, deg1, b, w):
    n = g.shape[0]
    return pl.pallas_call(
        _mid_body,
        grid=(n // _BM,),
        in_specs=[
            pl.BlockSpec((_BM, D), lambda i: (i, 0)),
            pl.BlockSpec((_BM, D), lambda i: (i, 0)),
            pl.BlockSpec((_BM, D), lambda i: (i, 0)),
            pl.BlockSpec((_BM, 1), lambda i: (i, 0)),
            pl.BlockSpec((_BM, 1), lambda i: (i, 0)),
            pl.BlockSpec((1, D), lambda i: (0, 0)),
            pl.BlockSpec((D, D), lambda i: (0, 0)),
        ],
        out_specs=pl.BlockSpec((_BM, D), lambda i: (i, 0)),
        out_shape=jax.ShapeDtypeStruct((n, D), jnp.float32),
    )(p0, p1, g, deg0, deg1, b, w)


def _final_body(p0_ref, p1_ref, g_ref, d0_ref, d1_ref, b_ref, o_ref):
    dinv = lax.rsqrt(d0_ref[...] + d1_ref[...] + 1.0)
    o_ref[...] = (p0_ref[...] + p1_ref[...] + g_ref[...]) * dinv + b_ref[...]


def _tc_final(p0, p1, g, deg0, deg1, b):
    n = g.shape[0]
    return pl.pallas_call(
        _final_body,
        grid=(n // _BM,),
        in_specs=[
            pl.BlockSpec((_BM, D), lambda i: (i, 0)),
            pl.BlockSpec((_BM, D), lambda i: (i, 0)),
            pl.BlockSpec((_BM, D), lambda i: (i, 0)),
            pl.BlockSpec((_BM, 1), lambda i: (i, 0)),
            pl.BlockSpec((_BM, 1), lambda i: (i, 0)),
            pl.BlockSpec((1, D), lambda i: (0, 0)),
        ],
        out_specs=pl.BlockSpec((_BM, D), lambda i: (i, 0)),
        out_shape=jax.ShapeDtypeStruct((n, D), jnp.float32),
    )(p0, p1, g, deg0, deg1, b)


def kernel(x, edge_index, W1, b1, W2, b2):
    n, d = x.shape
    e = edge_index.shape[1]

    ei = edge_index.astype(jnp.int32)
    src, dst = ei[0], ei[1]

    per_tile_chunk = -(-e // (NW * CHUNK))
    per_tile_chunk += per_tile_chunk % 2
    e_pad = per_tile_chunk * NW * CHUNK
    pad = e_pad - e
    src_pad = jnp.concatenate([src, jnp.zeros((pad,), jnp.int32)])
    dst_pad = jnp.concatenate([dst, jnp.full((pad,), n, jnp.int32)])
    src3d = src_pad.reshape(NW, per_tile_chunk, CHUNK)
    dst3d = dst_pad.reshape(NW, per_tile_chunk, CHUNK)

    degw = _sc_degree(dst3d, per_tile_chunk)
    deg0 = degw[:n].reshape(n, 1)
    deg1 = degw[N_PAD:N_PAD + n].reshape(n, 1)

    h1 = _tc_matmul(x, W1)
    g1 = _tc_scale(h1, deg0, deg1)

    p1 = _sc_segment_sum(g1, src3d, dst_pad, per_tile_chunk)
    g2 = _tc_mid(p1[0, :n], p1[1, :n], g1, deg0, deg1,
                 b1.reshape(1, d), W2)

    p2 = _sc_segment_sum(g2, src3d, dst_pad, per_tile_chunk)
    out = _tc_final(p2[0, :n], p2[1, :n], g2, deg0, deg1,
                    b2.reshape(1, d))
    return out


# speedup vs baseline: 32.5673x; 3.4664x over previous
import functools

import jax
import jax.numpy as jnp
from jax import lax
from jax.experimental import pallas as pl
from jax.experimental.pallas import tpu as pltpu
from jax.experimental.pallas import tpu_sc as plsc

N_NODES = 10000
D = 128

NC = 2
NS = 16
NW = NC * NS

CHUNK = 128
ROWS_PER_TILE = 640
N_PAD = ROWS_PER_TILE * NS

_MESH = plsc.VectorSubcoreMesh(core_axis_name="c", subcore_axis_name="s",
                               num_cores=NC, num_subcores=NS)


def _zero_fill(buf, nrows, ncols):
    zeros16 = jnp.zeros((16,), jnp.float32)

    def row(i, _):
        for j in range(ncols // 16):
            buf[i, pl.ds(j * 16, 16)] = zeros16
        return 0

    lax.fori_loop(0, nrows, row, 0)


def _copy_zero_to_slice(zbuf, dst, row0, nrows):
    zrows = zbuf.shape[0]
    off = 0
    while off < nrows:
        n = min(zrows, nrows - off)
        pltpu.sync_copy(zbuf.at[pl.ds(0, n)], dst.at[pl.ds(row0 + off, n)])
        off += n


def _sc_degree_body(dst_hbm, out_hbm, dsts_v, hist_v, red_v, stage_sh, sem,
                    *, chunks_per_tile):
    c = lax.axis_index("c")
    s = lax.axis_index("s")
    wid = c * NS + s

    cp = pltpu.async_copy(dst_hbm.at[wid], dsts_v, sem)

    def zrow(i, _):
        hist_v[pl.ds(i * 16, 16)] = jnp.zeros((16,), jnp.float32)
        return 0

    lax.fori_loop(0, N_PAD // 16, zrow, 0)
    cp.wait()

    ones16 = jnp.ones((16,), jnp.float32)

    def body(i, _):
        for j in range(CHUNK // 16):
            plsc.addupdate_scatter(hist_v, [dsts_v[i, pl.ds(j * 16, 16)]],
                                   ones16)
        return 0

    lax.fori_loop(0, chunks_per_tile, body, 0)

    pltpu.sync_copy(hist_v, stage_sh.at[pl.ds(s * N_PAD, N_PAD)])
    plsc.subcore_barrier()

    row0 = s * ROWS_PER_TILE
    for k in range(NS):
        pltpu.sync_copy(stage_sh.at[pl.ds(k * N_PAD + row0, ROWS_PER_TILE)],
                        red_v.at[pl.ds(k * ROWS_PER_TILE, ROWS_PER_TILE)])

    def red_row(j, _):
        tot = red_v[pl.ds(j * 16, 16)]
        for k in range(1, NS):
            tot = tot + red_v[pl.ds(k * ROWS_PER_TILE + j * 16, 16)]
        hist_v[pl.ds(j * 16, 16)] = tot
        return 0

    lax.fori_loop(0, ROWS_PER_TILE // 16, red_row, 0)
    pltpu.sync_copy(hist_v.at[pl.ds(0, ROWS_PER_TILE)],
                    out_hbm.at[pl.ds(c * N_PAD + row0, ROWS_PER_TILE)])


def _sc_segment_sum_body(g_hbm, src_hbm, dst_hbm, out_hbm,
                         srcs_v, dst0_v, dst1_v, rows0_v, rows1_v, acc_sh,
                         sem_i, sem0, sem1, semd0, semd1,
                         *, chunks_per_tile):
    c = lax.axis_index("c")
    s = lax.axis_index("s")
    wid = c * NS + s

    cp_s = pltpu.async_copy(src_hbm.at[wid], srcs_v, sem_i)

    _zero_fill(rows0_v, CHUNK, D)
    _copy_zero_to_slice(rows0_v, acc_sh, s * ROWS_PER_TILE, ROWS_PER_TILE)
    cp_s.wait()
    plsc.subcore_barrier()

    ebase = wid * chunks_per_tile * CHUNK

    pltpu.async_copy(dst_hbm.at[pl.ds(ebase, CHUNK)], dst0_v, semd0)
    pltpu.async_copy(dst_hbm.at[pl.ds(ebase + CHUNK, CHUNK)], dst1_v, semd1)
    pltpu.async_copy(g_hbm.at[srcs_v.at[0]], rows0_v, sem0)

    def pair(j, _):
        i0 = 2 * j
        pltpu.async_copy(g_hbm.at[srcs_v.at[i0 + 1]], rows1_v, sem1)
        pltpu.make_async_copy(dst_hbm.at[pl.ds(ebase, CHUNK)], dst0_v,
                              semd0).wait()
        pltpu.make_async_copy(g_hbm.at[srcs_v.at[i0]], rows0_v, sem0).wait()
        pltpu.sync_copy(rows0_v, acc_sh.at[dst0_v], add=True)

        @pl.when(i0 + 2 < chunks_per_tile)
        def _():
            pltpu.async_copy(g_hbm.at[srcs_v.at[i0 + 2]], rows0_v, sem0)
            pltpu.async_copy(dst_hbm.at[pl.ds(ebase + (i0 + 2) * CHUNK, CHUNK)],
                             dst0_v, semd0)

        pltpu.make_async_copy(dst_hbm.at[pl.ds(ebase, CHUNK)], dst1_v,
                              semd1).wait()
        pltpu.make_async_copy(g_hbm.at[srcs_v.at[i0 + 1]], rows1_v, sem1).wait()
        pltpu.sync_copy(rows1_v, acc_sh.at[dst1_v], add=True)

        @pl.when(i0 + 3 < chunks_per_tile)
        def _():
            pltpu.async_copy(dst_hbm.at[pl.ds(ebase + (i0 + 3) * CHUNK, CHUNK)],
                             dst1_v, semd1)

        return 0

    lax.fori_loop(0, chunks_per_tile // 2, pair, 0)
    plsc.subcore_barrier()

    row0 = s * ROWS_PER_TILE
    pltpu.sync_copy(acc_sh.at[pl.ds(row0, ROWS_PER_TILE)],
                    out_hbm.at[c].at[pl.ds(row0, ROWS_PER_TILE)])


def _sc_degree(dst_pad, chunks_per_tile):
    kfn = functools.partial(
        pl.kernel,
        out_type=jax.ShapeDtypeStruct((NC * N_PAD,), jnp.float32),
        mesh=_MESH,
        compiler_params=pltpu.CompilerParams(needs_layout_passes=False),
        scratch_types=[
            pltpu.VMEM((chunks_per_tile, CHUNK), jnp.int32),
            pltpu.VMEM((N_PAD,), jnp.float32),
            pltpu.VMEM((NS * ROWS_PER_TILE,), jnp.float32),
            pltpu.VMEM_SHARED((NS * N_PAD,), jnp.float32),
            pltpu.SemaphoreType.DMA,
        ],
    )(functools.partial(_sc_degree_body, chunks_per_tile=chunks_per_tile))
    return kfn(dst_pad)


def _sc_segment_sum(g, src3d, dst_flat, chunks_per_tile):
    kfn = functools.partial(
        pl.kernel,
        out_type=jax.ShapeDtypeStruct((NC, N_PAD, D), jnp.float32),
        mesh=_MESH,
        scratch_types=[
            pltpu.VMEM((chunks_per_tile, CHUNK), jnp.int32),
            pltpu.VMEM((CHUNK,), jnp.int32),
            pltpu.VMEM((CHUNK,), jnp.int32),
            pltpu.VMEM((CHUNK, D), jnp.float32),
            pltpu.VMEM((CHUNK, D), jnp.float32),
            pltpu.VMEM_SHARED((N_PAD, D), jnp.float32),
            pltpu.SemaphoreType.DMA,
            pltpu.SemaphoreType.DMA,
            pltpu.SemaphoreType.DMA,
            pltpu.SemaphoreType.DMA,
            pltpu.SemaphoreType.DMA,
        ],
    )(functools.partial(_sc_segment_sum_body, chunks_per_tile=chunks_per_tile))
    return kfn(g, src3d, dst_flat)


_BM = 1000


def _mm_body(x_ref, w_ref, o_ref):
    o_ref[...] = jnp.dot(x_ref[...], w_ref[...],
                         preferred_element_type=jnp.float32)


def _tc_matmul(x, w):
    n = x.shape[0]
    return pl.pallas_call(
        _mm_body,
        grid=(n // _BM,),
        in_specs=[
            pl.BlockSpec((_BM, D), lambda i: (i, 0)),
            pl.BlockSpec((D, D), lambda i: (0, 0)),
        ],
        out_specs=pl.BlockSpec((_BM, D), lambda i: (i, 0)),
        out_shape=jax.ShapeDtypeStruct((n, D), jnp.float32),
    )(x, w)


def _scale_body(h_ref, d0_ref, d1_ref, g_ref):
    dinv = lax.rsqrt(d0_ref[...] + d1_ref[...] + 1.0)
    g_ref[...] = h_ref[...] * dinv


def _tc_scale(h, deg0, deg1):
    n = h.shape[0]
    return pl.pallas_call(
        _scale_body,
        grid=(n // _BM,),
        in_specs=[
            pl.BlockSpec((_BM, D), lambda i: (i, 0)),
            pl.BlockSpec((_BM, 1), lambda i: (i, 0)),
            pl.BlockSpec((_BM, 1), lambda i: (i, 0)),
        ],
        out_specs=pl.BlockSpec((_BM, D), lambda i: (i, 0)),
        out_shape=jax.ShapeDtypeStruct((n, D), jnp.float32),
    )(h, deg0, deg1)


def _mid_body(p0_ref, p1_ref, g_ref, d0_ref, d1_ref, b_ref, w_ref, o_ref):
    dinv = lax.rsqrt(d0_ref[...] + d1_ref[...] + 1.0)
    t = (p0_ref[...] + p1_ref[...] + g_ref[...]) * dinv + b_ref[...]
    t = jnp.maximum(t, 0.0)
    o_ref[...] = jnp.dot(t, w_ref[...],
                         preferred_element_type=jnp.float32) * dinv


def _tc_mid(p0, p1, g, deg0, deg1, b, w):
    n = g.shape[0]
    return pl.pallas_call(
        _mid_body,
        grid=(n // _BM,),
        in_specs=[
            pl.BlockSpec((_BM, D), lambda i: (i, 0)),
            pl.BlockSpec((_BM, D), lambda i: (i, 0)),
            pl.BlockSpec((_BM, D), lambda i: (i, 0)),
            pl.BlockSpec((_BM, 1), lambda i: (i, 0)),
            pl.BlockSpec((_BM, 1), lambda i: (i, 0)),
            pl.BlockSpec((1, D), lambda i: (0, 0)),
            pl.BlockSpec((D, D), lambda i: (0, 0)),
        ],
        out_specs=pl.BlockSpec((_BM, D), lambda i: (i, 0)),
        out_shape=jax.ShapeDtypeStruct((n, D), jnp.float32),
    )(p0, p1, g, deg0, deg1, b, w)


def _final_body(p0_ref, p1_ref, g_ref, d0_ref, d1_ref, b_ref, o_ref):
    dinv = lax.rsqrt(d0_ref[...] + d1_ref[...] + 1.0)
    o_ref[...] = (p0_ref[...] + p1_ref[...] + g_ref[...]) * dinv + b_ref[...]


def _tc_final(p0, p1, g, deg0, deg1, b):
    n = g.shape[0]
    return pl.pallas_call(
        _final_body,
        grid=(n // _BM,),
        in_specs=[
            pl.BlockSpec((_BM, D), lambda i: (i, 0)),
            pl.BlockSpec((_BM, D), lambda i: (i, 0)),
            pl.BlockSpec((_BM, D), lambda i: (i, 0)),
            pl.BlockSpec((_BM, 1), lambda i: (i, 0)),
            pl.BlockSpec((_BM, 1), lambda i: (i, 0)),
            pl.BlockSpec((1, D), lambda i: (0, 0)),
        ],
        out_specs=pl.BlockSpec((_BM, D), lambda i: (i, 0)),
        out_shape=jax.ShapeDtypeStruct((n, D), jnp.float32),
    )(p0, p1, g, deg0, deg1, b)


def kernel(x, edge_index, W1, b1, W2, b2):
    n, d = x.shape
    e = edge_index.shape[1]

    ei = edge_index.astype(jnp.int32)
    src, dst = ei[0], ei[1]

    per_tile_chunk = -(-e // (NW * CHUNK))
    per_tile_chunk += per_tile_chunk % 2
    e_pad = per_tile_chunk * NW * CHUNK
    pad = e_pad - e
    cyc = jnp.arange(pad, dtype=jnp.int32)
    src_pad = jnp.concatenate([src, cyc % n])
    dst_pad = jnp.concatenate([dst, n + cyc % (N_PAD - n)])
    src3d = src_pad.reshape(NW, per_tile_chunk, CHUNK)
    dst3d = dst_pad.reshape(NW, per_tile_chunk, CHUNK)

    degw = _sc_degree(dst3d, per_tile_chunk)
    deg0 = degw[:n].reshape(n, 1)
    deg1 = degw[N_PAD:N_PAD + n].reshape(n, 1)

    h1 = _tc_matmul(x, W1)
    g1 = _tc_scale(h1, deg0, deg1)

    p1 = _sc_segment_sum(g1, src3d, dst_pad, per_tile_chunk)
    g2 = _tc_mid(p1[0, :n], p1[1, :n], g1, deg0, deg1,
                 b1.reshape(1, d), W2)

    p2 = _sc_segment_sum(g2, src3d, dst_pad, per_tile_chunk)
    out = _tc_final(p2[0, :n], p2[1, :n], g2, deg0, deg1,
                    b2.reshape(1, d))
    return out


# speedup vs baseline: 34.5895x; 1.0621x over previous
import functools

import jax
import jax.numpy as jnp
from jax import lax
from jax.experimental import pallas as pl
from jax.experimental.pallas import tpu as pltpu
from jax.experimental.pallas import tpu_sc as plsc

N_NODES = 10000
D = 128

NC = 2
NS = 16
NW = NC * NS

CHUNK = 64
NBUF = 4
ROWS_PER_TILE = 640
N_PAD = ROWS_PER_TILE * NS

_MESH = plsc.VectorSubcoreMesh(core_axis_name="c", subcore_axis_name="s",
                               num_cores=NC, num_subcores=NS)


def _zero_fill(buf, nrows, ncols):
    zeros16 = jnp.zeros((16,), jnp.float32)

    def row(i, _):
        for j in range(ncols // 16):
            buf[i, pl.ds(j * 16, 16)] = zeros16
        return 0

    lax.fori_loop(0, nrows, row, 0)


def _copy_zero_to_slice(zbuf, dst, row0, nrows):
    zrows = zbuf.shape[0]
    off = 0
    while off < nrows:
        n = min(zrows, nrows - off)
        pltpu.sync_copy(zbuf.at[pl.ds(0, n)], dst.at[pl.ds(row0 + off, n)])
        off += n


def _sc_degree_body(dst_hbm, out_hbm, dsts_v, hist_v, red_v, stage_sh, sem,
                    *, chunks_per_tile):
    c = lax.axis_index("c")
    s = lax.axis_index("s")
    wid = c * NS + s

    cp = pltpu.async_copy(dst_hbm.at[wid], dsts_v, sem)

    def zrow(i, _):
        hist_v[pl.ds(i * 16, 16)] = jnp.zeros((16,), jnp.float32)
        return 0

    lax.fori_loop(0, N_PAD // 16, zrow, 0)
    cp.wait()

    ones16 = jnp.ones((16,), jnp.float32)

    def body(i, _):
        for j in range(CHUNK // 16):
            plsc.addupdate_scatter(hist_v, [dsts_v[i, pl.ds(j * 16, 16)]],
                                   ones16)
        return 0

    lax.fori_loop(0, chunks_per_tile, body, 0)

    pltpu.sync_copy(hist_v, stage_sh.at[pl.ds(s * N_PAD, N_PAD)])
    plsc.subcore_barrier()

    row0 = s * ROWS_PER_TILE
    for k in range(NS):
        pltpu.sync_copy(stage_sh.at[pl.ds(k * N_PAD + row0, ROWS_PER_TILE)],
                        red_v.at[pl.ds(k * ROWS_PER_TILE, ROWS_PER_TILE)])

    def red_row(j, _):
        tot = red_v[pl.ds(j * 16, 16)]
        for k in range(1, NS):
            tot = tot + red_v[pl.ds(k * ROWS_PER_TILE + j * 16, 16)]
        hist_v[pl.ds(j * 16, 16)] = tot
        return 0

    lax.fori_loop(0, ROWS_PER_TILE // 16, red_row, 0)
    pltpu.sync_copy(hist_v.at[pl.ds(0, ROWS_PER_TILE)],
                    out_hbm.at[pl.ds(c * N_PAD + row0, ROWS_PER_TILE)])


def _sc_segment_sum_body(g_hbm, src_hbm, dst_hbm, out_hbm,
                         src_v, dst_v, rows_v, acc_sh, isem, jsem, gsem, ssem,
                         *, chunks_per_tile):
    c = lax.axis_index("c")
    s = lax.axis_index("s")
    wid = c * NS + s
    n = chunks_per_tile
    ebase = wid * n * CHUNK

    def src_load(i, b):
        pltpu.async_copy(src_hbm.at[pl.ds(ebase + i * CHUNK, CHUNK)],
                         src_v[b], isem[b])

    def src_wait(b):
        pltpu.make_async_copy(src_hbm.at[pl.ds(ebase, CHUNK)],
                              src_v[b], isem[b]).wait()

    def dst_load(i, b):
        pltpu.async_copy(dst_hbm.at[pl.ds(ebase + i * CHUNK, CHUNK)],
                         dst_v[b], jsem[b])

    def dst_wait(b):
        pltpu.make_async_copy(dst_hbm.at[pl.ds(ebase, CHUNK)],
                              dst_v[b], jsem[b]).wait()

    def gather(b):
        pltpu.async_copy(g_hbm.at[src_v[b]], rows_v[b], gsem[b])

    def gather_wait(b):
        pltpu.make_async_copy(g_hbm.at[src_v[b]], rows_v[b], gsem[b]).wait()

    def scatter(b):
        pltpu.async_copy(rows_v[b], acc_sh.at[dst_v[b]], ssem[b], add=True)

    def scatter_wait(b):
        pltpu.make_async_copy(rows_v[b], acc_sh.at[dst_v[b]], ssem[b]).wait()

    for b in range(NBUF):
        src_load(b, b)
    for b in range(2):
        dst_load(b, b)

    _zero_fill(rows_v[0], CHUNK, D)
    _zero_fill(rows_v[1], CHUNK, D)
    _copy_zero_to_slice(rows_v[0], acc_sh, s * ROWS_PER_TILE,
                        ROWS_PER_TILE // 2)
    _copy_zero_to_slice(rows_v[1], acc_sh,
                        s * ROWS_PER_TILE + ROWS_PER_TILE // 2,
                        ROWS_PER_TILE // 2)
    plsc.subcore_barrier()

    for b in range(2):
        src_wait(b)
        gather(b)

    def group(j, _):
        for b in range(NBUF):
            k = NBUF * j + b
            b2 = (b + 2) % NBUF

            @pl.when(k + 2 < n)
            def _():
                @pl.when(k >= 2)
                def _():
                    scatter_wait(b2)

                dst_load(k + 2, b2)
                src_wait(b2)
                gather(b2)

            gather_wait(b)
            dst_wait(b)
            scatter(b)

            @pl.when(k + 4 < n)
            def _():
                src_load(k + 4, b)

        return 0

    lax.fori_loop(0, n // NBUF, group, 0)
    for b in range(NBUF):
        scatter_wait(b)
    plsc.subcore_barrier()

    row0 = s * ROWS_PER_TILE
    pltpu.sync_copy(acc_sh.at[pl.ds(row0, ROWS_PER_TILE)],
                    out_hbm.at[c].at[pl.ds(row0, ROWS_PER_TILE)])


def _sc_degree(dst_pad, chunks_per_tile):
    kfn = functools.partial(
        pl.kernel,
        out_type=jax.ShapeDtypeStruct((NC * N_PAD,), jnp.float32),
        mesh=_MESH,
        compiler_params=pltpu.CompilerParams(needs_layout_passes=False),
        scratch_types=[
            pltpu.VMEM((chunks_per_tile, CHUNK), jnp.int32),
            pltpu.VMEM((N_PAD,), jnp.float32),
            pltpu.VMEM((NS * ROWS_PER_TILE,), jnp.float32),
            pltpu.VMEM_SHARED((NS * N_PAD,), jnp.float32),
            pltpu.SemaphoreType.DMA,
        ],
    )(functools.partial(_sc_degree_body, chunks_per_tile=chunks_per_tile))
    return kfn(dst_pad)


def _sc_segment_sum(g, src_flat, dst_flat, chunks_per_tile):
    kfn = functools.partial(
        pl.kernel,
        out_type=jax.ShapeDtypeStruct((NC, N_PAD, D), jnp.float32),
        mesh=_MESH,
        scratch_types=[
            [pltpu.VMEM((CHUNK,), jnp.int32)] * NBUF,
            [pltpu.VMEM((CHUNK,), jnp.int32)] * NBUF,
            [pltpu.VMEM((CHUNK, D), jnp.float32)] * NBUF,
            pltpu.VMEM_SHARED((N_PAD, D), jnp.float32),
            [pltpu.SemaphoreType.DMA] * NBUF,
            [pltpu.SemaphoreType.DMA] * NBUF,
            [pltpu.SemaphoreType.DMA] * NBUF,
            [pltpu.SemaphoreType.DMA] * NBUF,
        ],
    )(functools.partial(_sc_segment_sum_body, chunks_per_tile=chunks_per_tile))
    return kfn(g, src_flat, dst_flat)


_BM = 1000


def _mm_body(x_ref, w_ref, o_ref):
    o_ref[...] = jnp.dot(x_ref[...], w_ref[...],
                         preferred_element_type=jnp.float32)


def _tc_matmul(x, w):
    n = x.shape[0]
    return pl.pallas_call(
        _mm_body,
        grid=(n // _BM,),
        in_specs=[
            pl.BlockSpec((_BM, D), lambda i: (i, 0)),
            pl.BlockSpec((D, D), lambda i: (0, 0)),
        ],
        out_specs=pl.BlockSpec((_BM, D), lambda i: (i, 0)),
        out_shape=jax.ShapeDtypeStruct((n, D), jnp.float32),
    )(x, w)


def _scale_body(h_ref, d0_ref, d1_ref, g_ref):
    dinv = lax.rsqrt(d0_ref[...] + d1_ref[...] + 1.0)
    g_ref[...] = h_ref[...] * dinv


def _tc_scale(h, deg0, deg1):
    n = h.shape[0]
    return pl.pallas_call(
        _scale_body,
        grid=(n // _BM,),
        in_specs=[
            pl.BlockSpec((_BM, D), lambda i: (i, 0)),
            pl.BlockSpec((_BM, 1), lambda i: (i, 0)),
            pl.BlockSpec((_BM, 1), lambda i: (i, 0)),
        ],
        out_specs=pl.BlockSpec((_BM, D), lambda i: (i, 0)),
        out_shape=jax.ShapeDtypeStruct((n, D), jnp.float32),
    )(h, deg0, deg1)


def _mid_body(p_ref, g_ref, d0_ref, d1_ref, b_ref, w_ref, o_ref):
    dinv = lax.rsqrt(d0_ref[...] + d1_ref[...] + 1.0)
    t = (p_ref[0] + p_ref[1] + g_ref[...]) * dinv + b_ref[...]
    t = jnp.maximum(t, 0.0)
    o_ref[...] = jnp.dot(t, w_ref[...],
                         preferred_element_type=jnp.float32) * dinv


def _tc_mid(p, g, deg0, deg1, b, w):
    n = g.shape[0]
    return pl.pallas_call(
        _mid_body,
        grid=(n // _BM,),
        in_specs=[
            pl.BlockSpec((NC, _BM, D), lambda i: (0, i, 0)),
            pl.BlockSpec((_BM, D), lambda i: (i, 0)),
            pl.BlockSpec((_BM, 1), lambda i: (i, 0)),
            pl.BlockSpec((_BM, 1), lambda i: (i, 0)),
            pl.BlockSpec((1, D), lambda i: (0, 0)),
            pl.BlockSpec((D, D), lambda i: (0, 0)),
        ],
        out_specs=pl.BlockSpec((_BM, D), lambda i: (i, 0)),
        out_shape=jax.ShapeDtypeStruct((n, D), jnp.float32),
    )(p, g, deg0, deg1, b, w)


def _final_body(p_ref, g_ref, d0_ref, d1_ref, b_ref, o_ref):
    dinv = lax.rsqrt(d0_ref[...] + d1_ref[...] + 1.0)
    o_ref[...] = (p_ref[0] + p_ref[1] + g_ref[...]) * dinv + b_ref[...]


def _tc_final(p, g, deg0, deg1, b):
    n = g.shape[0]
    return pl.pallas_call(
        _final_body,
        grid=(n // _BM,),
        in_specs=[
            pl.BlockSpec((NC, _BM, D), lambda i: (0, i, 0)),
            pl.BlockSpec((_BM, D), lambda i: (i, 0)),
            pl.BlockSpec((_BM, 1), lambda i: (i, 0)),
            pl.BlockSpec((_BM, 1), lambda i: (i, 0)),
            pl.BlockSpec((1, D), lambda i: (0, 0)),
        ],
        out_specs=pl.BlockSpec((_BM, D), lambda i: (i, 0)),
        out_shape=jax.ShapeDtypeStruct((n, D), jnp.float32),
    )(p, g, deg0, deg1, b)


def kernel(x, edge_index, W1, b1, W2, b2):
    n, d = x.shape
    e = edge_index.shape[1]

    ei = edge_index.astype(jnp.int32)
    src, dst = ei[0], ei[1]

    per_tile_chunk = -(-e // (NW * CHUNK))
    per_tile_chunk = -(-per_tile_chunk // NBUF) * NBUF
    e_pad = per_tile_chunk * NW * CHUNK
    pad = e_pad - e
    cyc = jnp.arange(pad, dtype=jnp.int32)
    src_pad = jnp.concatenate([src, cyc % n])
    dst_pad = jnp.concatenate([dst, n + cyc % (N_PAD - n)])
    dst3d = dst_pad.reshape(NW, per_tile_chunk, CHUNK)

    degw = _sc_degree(dst3d, per_tile_chunk)
    deg0 = degw[:n].reshape(n, 1)
    deg1 = degw[N_PAD:N_PAD + n].reshape(n, 1)

    h1 = _tc_matmul(x, W1)
    g1 = _tc_scale(h1, deg0, deg1)

    p1 = _sc_segment_sum(g1, src_pad, dst_pad, per_tile_chunk)
    g2 = _tc_mid(p1, g1, deg0, deg1, b1.reshape(1, d), W2)

    p2 = _sc_segment_sum(g2, src_pad, dst_pad, per_tile_chunk)
    out = _tc_final(p2, g2, deg0, deg1, b2.reshape(1, d))
    return out


# speedup vs baseline: 36.2825x; 1.0489x over previous
import functools

import jax
import jax.numpy as jnp
from jax import lax
from jax.experimental import pallas as pl
from jax.experimental.pallas import tpu as pltpu
from jax.experimental.pallas import tpu_sc as plsc

N_NODES = 10000
D = 128

NC = 2
NS = 16
NW = NC * NS

CHUNK = 64
NBUF = 4
ROWS_PER_TILE = 640
N_PAD = ROWS_PER_TILE * NS

_MESH = plsc.VectorSubcoreMesh(core_axis_name="c", subcore_axis_name="s",
                               num_cores=NC, num_subcores=NS)


def _zero_fill(buf, nrows, ncols):
    zeros16 = jnp.zeros((16,), jnp.float32)

    def row(i, _):
        for j in range(ncols // 16):
            buf[i, pl.ds(j * 16, 16)] = zeros16
        return 0

    lax.fori_loop(0, nrows, row, 0)


def _copy_zero_to_slice(zbuf, dst, row0, nrows):
    zrows = zbuf.shape[0]
    off = 0
    while off < nrows:
        n = min(zrows, nrows - off)
        pltpu.sync_copy(zbuf.at[pl.ds(0, n)], dst.at[pl.ds(row0 + off, n)])
        off += n


def _sc_degree_body(edge_hbm, out_hbm, dsts_v, hist_v, red_v, stage_sh, sem,
                    *, n_edges, per_tile):
    c = lax.axis_index("c")
    s = lax.axis_index("s")
    wid = c * NS + s

    cp = pltpu.async_copy(edge_hbm.at[pl.ds(n_edges + wid * per_tile,
                                            per_tile)], dsts_v, sem)

    def zrow(i, _):
        hist_v[pl.ds(i * 16, 16)] = jnp.zeros((16,), jnp.float32)
        return 0

    lax.fori_loop(0, N_PAD // 16, zrow, 0)
    cp.wait()

    ones16 = jnp.ones((16,), jnp.float32)

    def body(i, _):
        plsc.addupdate_scatter(hist_v, [dsts_v[pl.ds(i * 16, 16)]], ones16)
        return 0

    lax.fori_loop(0, per_tile // 16, body, 0)

    pltpu.sync_copy(hist_v, stage_sh.at[pl.ds(s * N_PAD, N_PAD)])
    plsc.subcore_barrier()

    row0 = s * ROWS_PER_TILE
    for k in range(NS):
        pltpu.sync_copy(stage_sh.at[pl.ds(k * N_PAD + row0, ROWS_PER_TILE)],
                        red_v.at[pl.ds(k * ROWS_PER_TILE, ROWS_PER_TILE)])

    def red_row(j, _):
        tot = red_v[pl.ds(j * 16, 16)]
        for k in range(1, NS):
            tot = tot + red_v[pl.ds(k * ROWS_PER_TILE + j * 16, 16)]
        hist_v[pl.ds(j * 16, 16)] = tot
        return 0

    lax.fori_loop(0, ROWS_PER_TILE // 16, red_row, 0)
    pltpu.sync_copy(hist_v.at[pl.ds(0, ROWS_PER_TILE)],
                    out_hbm.at[pl.ds(c * N_PAD + row0, ROWS_PER_TILE)])


def _sc_segment_sum_body(g_hbm, edge_hbm, out_hbm,
                         src_v, dst_v, rows_v, srcr_v, dstr_v, rowsr_v,
                         acc_sh, isem, jsem, gsem, ssem,
                         *, n_edges, per_tile, main_chunks, rem):
    c = lax.axis_index("c")
    s = lax.axis_index("s")
    wid = c * NS + s
    n = main_chunks
    sbase = wid * per_tile
    dbase = n_edges + wid * per_tile

    def src_load(i, b):
        pltpu.async_copy(edge_hbm.at[pl.ds(sbase + i * CHUNK, CHUNK)],
                         src_v[b], isem[b])

    def src_wait(b):
        pltpu.make_async_copy(edge_hbm.at[pl.ds(sbase, CHUNK)],
                              src_v[b], isem[b]).wait()

    def dst_load(i, b):
        pltpu.async_copy(edge_hbm.at[pl.ds(dbase + i * CHUNK, CHUNK)],
                         dst_v[b], jsem[b])

    def dst_wait(b):
        pltpu.make_async_copy(edge_hbm.at[pl.ds(dbase, CHUNK)],
                              dst_v[b], jsem[b]).wait()

    def gather(b):
        pltpu.async_copy(g_hbm.at[src_v[b]], rows_v[b], gsem[b])

    def gather_wait(b):
        pltpu.make_async_copy(g_hbm.at[src_v[b]], rows_v[b], gsem[b]).wait()

    def scatter(b):
        pltpu.async_copy(rows_v[b], acc_sh.at[dst_v[b]], ssem[b], add=True)

    def scatter_wait(b):
        pltpu.make_async_copy(rows_v[b], acc_sh.at[dst_v[b]], ssem[b]).wait()

    for b in range(NBUF):
        src_load(b, b)
    for b in range(2):
        dst_load(b, b)

    _zero_fill(rows_v[0], CHUNK, D)
    _zero_fill(rows_v[1], CHUNK, D)
    _copy_zero_to_slice(rows_v[0], acc_sh, s * ROWS_PER_TILE,
                        ROWS_PER_TILE // 2)
    _copy_zero_to_slice(rows_v[1], acc_sh,
                        s * ROWS_PER_TILE + ROWS_PER_TILE // 2,
                        ROWS_PER_TILE // 2)
    plsc.subcore_barrier()

    for b in range(2):
        src_wait(b)
        gather(b)

    def group(j, _):
        for b in range(NBUF):
            k = NBUF * j + b
            b2 = (b + 2) % NBUF

            @pl.when(k + 2 < n)
            def _():
                @pl.when(k >= 2)
                def _():
                    scatter_wait(b2)

                dst_load(k + 2, b2)
                src_wait(b2)
                gather(b2)

            gather_wait(b)
            dst_wait(b)
            scatter(b)

            @pl.when(k + 4 < n)
            def _():
                src_load(k + 4, b)

        return 0

    lax.fori_loop(0, n // NBUF, group, 0)
    for b in range(NBUF):
        scatter_wait(b)

    if rem:
        nmain = n * CHUNK
        pltpu.sync_copy(edge_hbm.at[pl.ds(sbase + nmain, rem)], srcr_v)
        pltpu.sync_copy(edge_hbm.at[pl.ds(dbase + nmain, rem)], dstr_v)
        pltpu.async_copy(g_hbm.at[srcr_v], rowsr_v, gsem[0]).wait()
        pltpu.sync_copy(rowsr_v, acc_sh.at[dstr_v], add=True)

    plsc.subcore_barrier()

    row0 = s * ROWS_PER_TILE
    pltpu.sync_copy(acc_sh.at[pl.ds(row0, ROWS_PER_TILE)],
                    out_hbm.at[c].at[pl.ds(row0, ROWS_PER_TILE)])


def _sc_degree(edge_flat, n_edges, per_tile):
    kfn = functools.partial(
        pl.kernel,
        out_type=jax.ShapeDtypeStruct((NC * N_PAD,), jnp.float32),
        mesh=_MESH,
        compiler_params=pltpu.CompilerParams(needs_layout_passes=False),
        scratch_types=[
            pltpu.VMEM((per_tile,), jnp.int32),
            pltpu.VMEM((N_PAD,), jnp.float32),
            pltpu.VMEM((NS * ROWS_PER_TILE,), jnp.float32),
            pltpu.VMEM_SHARED((NS * N_PAD,), jnp.float32),
            pltpu.SemaphoreType.DMA,
        ],
    )(functools.partial(_sc_degree_body, n_edges=n_edges, per_tile=per_tile))
    return kfn(edge_flat)


def _sc_segment_sum(g, edge_flat, n_edges, per_tile, main_chunks, rem):
    kfn = functools.partial(
        pl.kernel,
        out_type=jax.ShapeDtypeStruct((NC, N_PAD, D), jnp.float32),
        mesh=_MESH,
        scratch_types=[
            [pltpu.VMEM((CHUNK,), jnp.int32)] * NBUF,
            [pltpu.VMEM((CHUNK,), jnp.int32)] * NBUF,
            [pltpu.VMEM((CHUNK, D), jnp.float32)] * NBUF,
            pltpu.VMEM((max(rem, 8),), jnp.int32),
            pltpu.VMEM((max(rem, 8),), jnp.int32),
            pltpu.VMEM((max(rem, 8), D), jnp.float32),
            pltpu.VMEM_SHARED((N_PAD, D), jnp.float32),
            [pltpu.SemaphoreType.DMA] * NBUF,
            [pltpu.SemaphoreType.DMA] * NBUF,
            [pltpu.SemaphoreType.DMA] * NBUF,
            [pltpu.SemaphoreType.DMA] * NBUF,
        ],
    )(functools.partial(_sc_segment_sum_body, n_edges=n_edges,
                        per_tile=per_tile, main_chunks=main_chunks, rem=rem))
    return kfn(g, edge_flat)


_BM = 1000


def _mm_body(x_ref, w_ref, o_ref):
    o_ref[...] = jnp.dot(x_ref[...], w_ref[...],
                         preferred_element_type=jnp.float32)


def _tc_matmul(x, w):
    n = x.shape[0]
    return pl.pallas_call(
        _mm_body,
        grid=(n // _BM,),
        in_specs=[
            pl.BlockSpec((_BM, D), lambda i: (i, 0)),
            pl.BlockSpec((D, D), lambda i: (0, 0)),
        ],
        out_specs=pl.BlockSpec((_BM, D), lambda i: (i, 0)),
        out_shape=jax.ShapeDtypeStruct((n, D), jnp.float32),
    )(x, w)


def _scale_body(h_ref, d0_ref, d1_ref, g_ref):
    dinv = lax.rsqrt(d0_ref[...] + d1_ref[...] + 1.0)
    g_ref[...] = h_ref[...] * dinv


def _tc_scale(h, deg0, deg1):
    n = h.shape[0]
    return pl.pallas_call(
        _scale_body,
        grid=(n // _BM,),
        in_specs=[
            pl.BlockSpec((_BM, D), lambda i: (i, 0)),
            pl.BlockSpec((_BM, 1), lambda i: (i, 0)),
            pl.BlockSpec((_BM, 1), lambda i: (i, 0)),
        ],
        out_specs=pl.BlockSpec((_BM, D), lambda i: (i, 0)),
        out_shape=jax.ShapeDtypeStruct((n, D), jnp.float32),
    )(h, deg0, deg1)


def _mid_body(p_ref, g_ref, d0_ref, d1_ref, b_ref, w_ref, o_ref):
    dinv = lax.rsqrt(d0_ref[...] + d1_ref[...] + 1.0)
    t = (p_ref[0] + p_ref[1] + g_ref[...]) * dinv + b_ref[...]
    t = jnp.maximum(t, 0.0)
    o_ref[...] = jnp.dot(t, w_ref[...],
                         preferred_element_type=jnp.float32) * dinv


def _tc_mid(p, g, deg0, deg1, b, w):
    n = g.shape[0]
    return pl.pallas_call(
        _mid_body,
        grid=(n // _BM,),
        in_specs=[
            pl.BlockSpec((NC, _BM, D), lambda i: (0, i, 0)),
            pl.BlockSpec((_BM, D), lambda i: (i, 0)),
            pl.BlockSpec((_BM, 1), lambda i: (i, 0)),
            pl.BlockSpec((_BM, 1), lambda i: (i, 0)),
            pl.BlockSpec((1, D), lambda i: (0, 0)),
            pl.BlockSpec((D, D), lambda i: (0, 0)),
        ],
        out_specs=pl.BlockSpec((_BM, D), lambda i: (i, 0)),
        out_shape=jax.ShapeDtypeStruct((n, D), jnp.float32),
    )(p, g, deg0, deg1, b, w)


def _final_body(p_ref, g_ref, d0_ref, d1_ref, b_ref, o_ref):
    dinv = lax.rsqrt(d0_ref[...] + d1_ref[...] + 1.0)
    o_ref[...] = (p_ref[0] + p_ref[1] + g_ref[...]) * dinv + b_ref[...]


def _tc_final(p, g, deg0, deg1, b):
    n = g.shape[0]
    return pl.pallas_call(
        _final_body,
        grid=(n // _BM,),
        in_specs=[
            pl.BlockSpec((NC, _BM, D), lambda i: (0, i, 0)),
            pl.BlockSpec((_BM, D), lambda i: (i, 0)),
            pl.BlockSpec((_BM, 1), lambda i: (i, 0)),
            pl.BlockSpec((_BM, 1), lambda i: (i, 0)),
            pl.BlockSpec((1, D), lambda i: (0, 0)),
        ],
        out_specs=pl.BlockSpec((_BM, D), lambda i: (i, 0)),
        out_shape=jax.ShapeDtypeStruct((n, D), jnp.float32),
    )(p, g, deg0, deg1, b)


def kernel(x, edge_index, W1, b1, W2, b2):
    n, d = x.shape
    e = edge_index.shape[1]

    edge_flat = edge_index.astype(jnp.int32).reshape(2 * e)
    assert e % NW == 0
    per_tile = e // NW
    assert per_tile % 16 == 0
    main_chunks = per_tile // CHUNK // NBUF * NBUF
    rem = per_tile - main_chunks * CHUNK

    degw = _sc_degree(edge_flat, e, per_tile)
    deg0 = degw[:n].reshape(n, 1)
    deg1 = degw[N_PAD:N_PAD + n].reshape(n, 1)

    h1 = _tc_matmul(x, W1)
    g1 = _tc_scale(h1, deg0, deg1)

    p1 = _sc_segment_sum(g1, edge_flat, e, per_tile, main_chunks, rem)
    g2 = _tc_mid(p1, g1, deg0, deg1, b1.reshape(1, d), W2)

    p2 = _sc_segment_sum(g2, edge_flat, e, per_tile, main_chunks, rem)
    out = _tc_final(p2, g2, deg0, deg1, b2.reshape(1, d))
    return out
